# Initial kernel scaffold; baseline (speedup 1.0000x reference)
import jax
import jax.numpy as jnp
from jax.experimental import pallas as pl


def kernel(x, edge_index, edge_feats, edge_types, W_iou_w, W_iou_b, U_iou_w, U_iou_b, W_f_w, W_f_b, U_f_w, U_f_b):
    raise NotImplementedError("write your pallas kernel here")


# speedup vs baseline: 8.4896x; 8.4896x over previous
import jax
import jax.numpy as jnp
from jax import lax
from jax.experimental import pallas as pl
from jax.experimental.pallas import tpu as pltpu

_N = 10000
_E = 160000
_F = 128
_F3 = 384


def _sigmoid(v):
    return 0.5 * (jnp.tanh(0.5 * v) + 1.0)


def _body(pe_ref, nperm_ref, ndeg_ref, niter_ref,
          x_ref, WiouT_ref, biou_ref, UiouT_ref, buiou_ref,
          WfT_ref, bfsum_ref, UfT_ref,
          h_ref,
          xfb_ref, c_ref, hsum_ref, hUf_ref, obuf_ref, fcbuf_ref):
    h_ref[...] = jnp.zeros((_N, _F), jnp.float32)
    c_ref[...] = jnp.zeros((_N, _F), jnp.float32)
    hsum_ref[...] = jnp.zeros((_N, _F), jnp.float32)
    hUf_ref[...] = jnp.zeros((_N, _F), jnp.float32)
    xfb_ref[...] = (
        jnp.dot(x_ref[...], WfT_ref[...], preferred_element_type=jnp.float32)
        + bfsum_ref[...]
    )

    def iter_body(r, carry):
        eptr, nptr = carry

        def a_cond(e):
            packed = pe_ref[jnp.minimum(e, _E - 1)]
            is_start = packed >> 28
            return (e < _E) & ((e == eptr) | (is_start == 0))

        def a_body(e):
            packed = pe_ref[e]
            p = (packed >> 14) & 16383
            ch = packed & 16383
            f = _sigmoid(xfb_ref[pl.ds(p, 1), :] + hUf_ref[pl.ds(ch, 1), :])
            fcbuf_ref[pl.ds(e - eptr, 1), :] = f * c_ref[pl.ds(ch, 1), :]
            hsum_ref[pl.ds(p, 1), :] = (
                hsum_ref[pl.ds(p, 1), :] + h_ref[pl.ds(ch, 1), :]
            )
            return e + 1

        eend = lax.cond(
            r > 0,
            lambda: lax.while_loop(a_cond, a_body, eptr),
            lambda: eptr,
        )

        def b_cond(q):
            return (q < _N) & (ndeg_ref[jnp.minimum(q, _N - 1)] == r)

        def b_body(q):
            v = nperm_ref[q]
            iou = (
                jnp.dot(x_ref[pl.ds(v, 1), :], WiouT_ref[...],
                        preferred_element_type=jnp.float32)
                + biou_ref[...]
            )
            term = (
                jnp.dot(hsum_ref[pl.ds(v, 1), :], UiouT_ref[...],
                        preferred_element_type=jnp.float32)
                + buiou_ref[...]
            )
            iou = iou + jnp.where(r > 0, 1.0, 0.0) * term
            gi = _sigmoid(iou[:, :_F])
            go = _sigmoid(iou[:, _F:2 * _F])
            gu = jnp.tanh(iou[:, 2 * _F:])
            c_ref[pl.ds(v, 1), :] = gi * gu
            obuf_ref[pl.ds(v, 1), :] = go
            return q + 1

        nend = lax.while_loop(b_cond, b_body, nptr)

        def c_body(j, _):
            packed = pe_ref[eptr + j]
            p = (packed >> 14) & 16383
            c_ref[pl.ds(p, 1), :] = (
                c_ref[pl.ds(p, 1), :] + fcbuf_ref[pl.ds(j, 1), :]
            )
            return 0

        lax.fori_loop(0, eend - eptr, c_body, 0)

        def d_body(q, _):
            v = nperm_ref[q]
            hrow = obuf_ref[pl.ds(v, 1), :] * jnp.tanh(c_ref[pl.ds(v, 1), :])
            h_ref[pl.ds(v, 1), :] = hrow
            hUf_ref[pl.ds(v, 1), :] = jnp.dot(
                hrow, UfT_ref[...], preferred_element_type=jnp.float32)
            return 0

        lax.fori_loop(nptr, nend, d_body, 0)
        return (eend, nend)

    lax.fori_loop(0, niter_ref[0], iter_body, (0, 0))


def kernel(x, edge_index, edge_feats, edge_types,
           W_iou_w, W_iou_b, U_iou_w, U_iou_b,
           W_f_w, W_f_b, U_f_w, U_f_b):
    del edge_feats, edge_types

    parents = edge_index[0].astype(jnp.int32)
    children = edge_index[1].astype(jnp.int32)
    order = jnp.argsort(parents, stable=True)
    sp = parents[order]
    idx = jnp.arange(_E, dtype=jnp.int32)
    is_grp = jnp.concatenate([jnp.zeros((1,), dtype=bool), sp[1:] != sp[:-1]])
    group_start = jnp.where(is_grp, idx, 0)
    group_start = lax.cummax(group_start, axis=0)
    pos = idx - group_start + 1
    rank = jnp.zeros(_E, dtype=jnp.int32).at[order].set(pos)
    deg = jnp.bincount(parents, length=_N).astype(jnp.int32)
    niter = (deg.max() + 1).astype(jnp.int32).reshape(1)

    rperm = jnp.argsort(rank, stable=True)
    rank_s = rank[rperm]
    p_s = parents[rperm]
    ch_s = children[rperm]
    is_start = jnp.concatenate(
        [jnp.ones((1,), dtype=bool), rank_s[1:] != rank_s[:-1]])
    packed = (
        is_start.astype(jnp.int32) * (1 << 28) + p_s * (1 << 14) + ch_s
    )

    nperm = jnp.argsort(deg, stable=True).astype(jnp.int32)
    ndeg = deg[nperm]

    smem = pl.BlockSpec(memory_space=pltpu.SMEM)
    vmem = pl.BlockSpec(memory_space=pltpu.VMEM)
    out = pl.pallas_call(
        _body,
        out_shape=jax.ShapeDtypeStruct((_N, _F), jnp.float32),
        in_specs=[smem, smem, smem, smem] + [vmem] * 8,
        out_specs=vmem,
        scratch_shapes=[
            pltpu.VMEM((_N, _F), jnp.float32),
            pltpu.VMEM((_N, _F), jnp.float32),
            pltpu.VMEM((_N, _F), jnp.float32),
            pltpu.VMEM((_N, _F), jnp.float32),
            pltpu.VMEM((_N, _F), jnp.float32),
            pltpu.VMEM((_N, _F), jnp.float32),
        ],
    )(
        packed, nperm, ndeg, niter,
        x,
        W_iou_w.T, W_iou_b.reshape(1, _F3),
        U_iou_w.T, U_iou_b.reshape(1, _F3),
        W_f_w.T, (W_f_b + U_f_b).reshape(1, _F),
        U_f_w.T,
    )
    return out
